# Initial kernel scaffold; baseline (speedup 1.0000x reference)
import jax
import jax.numpy as jnp
from jax.experimental import pallas as pl


def kernel(x, edge_index, W1, b1, W2, b2):
    raise NotImplementedError("write your pallas kernel here")


# speedup vs baseline: 8.8242x; 8.8242x over previous
import functools

import jax
import jax.numpy as jnp
from jax import lax
from jax.experimental import pallas as pl
from jax.experimental.pallas import tpu as pltpu
from jax.experimental.pallas import tpu_sc as plsc

N = 10000
D = 256
DH = 128
E = 160000
NC = 2
NS = 16
CHUNK = 128
E_PAD = 163840
NPAD = 10240
RPT = NPAD // NS
ROWB = 400
GRID_R = N // ROWB

_MESH = plsc.VectorSubcoreMesh(core_axis_name="c", subcore_axis_name="s")


def _zero_fill(buf, rows):
    width = buf.shape[1]
    z = jnp.zeros((16,), jnp.float32)

    @pl.loop(0, rows)
    def _(r):
        @pl.loop(0, width // 16)
        def _(c):
            buf[r, pl.ds(c * 16, 16)] = z


@functools.partial(
    pl.kernel,
    mesh=_MESH,
    out_type=jax.ShapeDtypeStruct((NC, NPAD, 16), jnp.float32),
    scratch_types=[
        pltpu.VMEM((1, CHUNK), jnp.int32),
        pltpu.VMEM((CHUNK, 16), jnp.float32),
        pltpu.VMEM((64, 16), jnp.float32),
        pltpu.VMEM_SHARED((NPAD, 16), jnp.float32),
    ],
)
def _deg_kernel(dst_hbm, out_hbm, dstb, ones, zbuf, dacc):
    cid = lax.axis_index("c")
    sid = lax.axis_index("s")

    _zero_fill(zbuf, 64)
    o = jnp.ones((16,), jnp.float32)

    @pl.loop(0, CHUNK)
    def _(r):
        ones[r, pl.ds(0, 16)] = o

    @pl.loop(0, RPT // 64)
    def _(k):
        pltpu.sync_copy(zbuf, dacc.at[pl.ds(sid * RPT + k * 64, 64)])

    plsc.subcore_barrier()

    n_chunks = E_PAD // (NC * NS * CHUNK)
    row0 = (cid * NS + sid) * n_chunks

    @pl.loop(0, n_chunks)
    def _(ci):
        pltpu.sync_copy(dst_hbm.at[pl.ds(row0 + ci, 1)], dstb)
        pltpu.sync_copy(ones, dacc.at[dstb.at[0]], add=True)

    plsc.subcore_barrier()
    pltpu.sync_copy(
        dacc.at[pl.ds(sid * RPT, RPT)], out_hbm.at[cid, pl.ds(sid * RPT, RPT)]
    )


@functools.partial(
    pl.kernel,
    mesh=_MESH,
    out_type=jax.ShapeDtypeStruct((NC, NPAD, DH), jnp.float32),
    scratch_types=[
        pltpu.VMEM((CHUNK,), jnp.int32),
        pltpu.VMEM((CHUNK,), jnp.int32),
        pltpu.VMEM((1, CHUNK), jnp.int32),
        pltpu.VMEM((CHUNK, DH), jnp.float32),
        pltpu.VMEM((64, DH), jnp.float32),
        pltpu.VMEM_SHARED((NPAD, DH), jnp.float32),
        pltpu.SemaphoreType.DMA,
    ],
)
def _agg_kernel(hp_hbm, src_hbm, dst_hbm, out_hbm,
                srcb, gsrcb, dstb, rows, zbuf, acc, sem):
    cid = lax.axis_index("c")
    sid = lax.axis_index("s")

    _zero_fill(zbuf, 64)

    @pl.loop(0, RPT // 64)
    def _(k):
        pltpu.sync_copy(zbuf, acc.at[pl.ds(sid * RPT + k * 64, 64)])

    plsc.subcore_barrier()

    n_chunks = E_PAD // (NS * CHUNK)
    row0 = sid * n_chunks
    off = cid * NPAD

    @pl.loop(0, n_chunks)
    def _(ci):
        e0 = (row0 + ci) * CHUNK
        pltpu.sync_copy(src_hbm.at[pl.ds(e0, CHUNK)], srcb)
        pltpu.sync_copy(dst_hbm.at[pl.ds(row0 + ci, 1)], dstb)

        @pl.loop(0, CHUNK // 16)
        def _(j):
            gsrcb[pl.ds(j * 16, 16)] = srcb[pl.ds(j * 16, 16)] + off

        pltpu.async_copy(hp_hbm.at[gsrcb], rows, sem).wait()
        pltpu.sync_copy(rows, acc.at[dstb.at[0]], add=True)

    plsc.subcore_barrier()
    pltpu.sync_copy(
        acc.at[pl.ds(sid * RPT, RPT)], out_hbm.at[cid, pl.ds(sid * RPT, RPT)]
    )


def _mm_body(x_ref, w_ref, o_ref):
    o_ref[0] = jnp.dot(x_ref[...], w_ref[...],
                       preferred_element_type=jnp.float32)


def _matmul_stacked(x, w):
    return pl.pallas_call(
        _mm_body,
        grid=(GRID_R, NC),
        in_specs=[
            pl.BlockSpec((ROWB, D), lambda i, j: (i, 0)),
            pl.BlockSpec((D, DH), lambda i, j: (0, j)),
        ],
        out_specs=pl.BlockSpec((1, ROWB, DH), lambda i, j: (j, i, 0)),
        out_shape=jax.ShapeDtypeStruct((NC, NPAD, DH), jnp.float32),
    )(x, w)


def _dinv_of(p0_ref, p1_ref):
    deg = p0_ref[0, :, :1] + p1_ref[0, :, :1] + 1.0
    return lax.rsqrt(deg)


def _scale_body(h_ref, p0_ref, p1_ref, o_ref):
    o_ref[0] = h_ref[0] * _dinv_of(p0_ref, p1_ref)


def _scale_stacked(h, degp):
    return pl.pallas_call(
        _scale_body,
        grid=(GRID_R, NC),
        in_specs=[
            pl.BlockSpec((1, ROWB, DH), lambda i, j: (j, i, 0)),
            pl.BlockSpec((1, ROWB, 16), lambda i, j: (0, i, 0)),
            pl.BlockSpec((1, ROWB, 16), lambda i, j: (1, i, 0)),
        ],
        out_specs=pl.BlockSpec((1, ROWB, DH), lambda i, j: (j, i, 0)),
        out_shape=jax.ShapeDtypeStruct((NC, NPAD, DH), jnp.float32),
    )(h, degp, degp)


def _layer2_body(a0_ref, a1_ref, h0_ref, h1_ref, p0_ref, p1_ref,
                 b_ref, w_ref, o_ref):
    dinv = _dinv_of(p0_ref, p1_ref)
    z0 = jnp.maximum(dinv * (a0_ref[0] + h0_ref[0]) + b_ref[:, :DH], 0.0)
    z1 = jnp.maximum(dinv * (a1_ref[0] + h1_ref[0]) + b_ref[:, DH:], 0.0)
    z = jnp.concatenate([z0, z1], axis=1)
    o_ref[0] = dinv * jnp.dot(z, w_ref[...],
                              preferred_element_type=jnp.float32)


def _layer2(acc1, hp1, degp, b1, w2):
    half = pl.BlockSpec((1, ROWB, DH), lambda i, j: (0, i, 0))
    half1 = pl.BlockSpec((1, ROWB, DH), lambda i, j: (1, i, 0))
    return pl.pallas_call(
        _layer2_body,
        grid=(GRID_R, NC),
        in_specs=[
            half, half1, half, half1,
            pl.BlockSpec((1, ROWB, 16), lambda i, j: (0, i, 0)),
            pl.BlockSpec((1, ROWB, 16), lambda i, j: (1, i, 0)),
            pl.BlockSpec((1, D), lambda i, j: (0, 0)),
            pl.BlockSpec((D, DH), lambda i, j: (0, j)),
        ],
        out_specs=pl.BlockSpec((1, ROWB, DH), lambda i, j: (j, i, 0)),
        out_shape=jax.ShapeDtypeStruct((NC, NPAD, DH), jnp.float32),
    )(acc1, acc1, hp1, hp1, degp, degp, b1, w2)


def _final_body(a_ref, h_ref, p0_ref, p1_ref, b_ref, o_ref):
    dinv = _dinv_of(p0_ref, p1_ref)
    o_ref[...] = dinv * (a_ref[0] + h_ref[0]) + b_ref[0]


def _final(acc2, hp2, degp, b2):
    return pl.pallas_call(
        _final_body,
        grid=(GRID_R, NC),
        in_specs=[
            pl.BlockSpec((1, ROWB, DH), lambda i, j: (j, i, 0)),
            pl.BlockSpec((1, ROWB, DH), lambda i, j: (j, i, 0)),
            pl.BlockSpec((1, ROWB, 16), lambda i, j: (0, i, 0)),
            pl.BlockSpec((1, ROWB, 16), lambda i, j: (1, i, 0)),
            pl.BlockSpec((1, 1, DH), lambda i, j: (j, 0, 0)),
        ],
        out_specs=pl.BlockSpec((ROWB, DH), lambda i, j: (i, j)),
        out_shape=jax.ShapeDtypeStruct((N, D), jnp.float32),
    )(acc2, hp2, degp, degp, b2)


def kernel(x, edge_index, W1, b1, W2, b2):
    src = edge_index[0].astype(jnp.int32)
    dst = edge_index[1].astype(jnp.int32)
    padi = jnp.arange(E_PAD - E, dtype=jnp.int32)
    src_p = jnp.concatenate([src, padi % N])
    dst_p = jnp.concatenate([dst, N + (padi % 64)])
    dst2d = dst_p.reshape(E_PAD // CHUNK, CHUNK)
    b1r = b1.reshape(1, D)
    b2r = b2.reshape(NC, 1, DH)

    h1 = _matmul_stacked(x, W1)
    degp = _deg_kernel(dst2d)
    hp1 = _scale_stacked(h1, degp)
    acc1 = _agg_kernel(hp1.reshape(NC * NPAD, DH), src_p, dst2d)
    hp2 = _layer2(acc1, hp1, degp, b1r, W2)
    acc2 = _agg_kernel(hp2.reshape(NC * NPAD, DH), src_p, dst2d)
    return _final(acc2, hp2, degp, b2r)


# speedup vs baseline: 14.3890x; 1.6306x over previous
import functools

import jax
import jax.numpy as jnp
from jax import lax
from jax.experimental import pallas as pl
from jax.experimental.pallas import tpu as pltpu
from jax.experimental.pallas import tpu_sc as plsc

N = 10000
D = 256
DH = 128
E = 160000
NC = 2
NS = 16
CHUNK = 128
E_PAD = 163840
NPAD = 10240
RPT = NPAD // NS
ROWB = 400
GRID_R = N // ROWB

_MESH = plsc.VectorSubcoreMesh(core_axis_name="c", subcore_axis_name="s")


def _zero_fill(buf, rows):
    width = buf.shape[1]
    z = jnp.zeros((16,), jnp.float32)

    @pl.loop(0, rows)
    def _(r):
        @pl.loop(0, width // 16)
        def _(c):
            buf[r, pl.ds(c * 16, 16)] = z


_DEG_CHUNKS = E_PAD // (NC * NS * CHUNK)


@functools.partial(
    pl.kernel,
    mesh=_MESH,
    out_type=jax.ShapeDtypeStruct((NC, NPAD, 16), jnp.float32),
    scratch_types=[
        pltpu.VMEM((_DEG_CHUNKS, 1, CHUNK), jnp.int32),
        pltpu.VMEM((CHUNK, 16), jnp.float32),
        pltpu.VMEM((64, 16), jnp.float32),
        pltpu.VMEM_SHARED((NPAD, 16), jnp.float32),
    ],
)
def _deg_kernel(dst_hbm, out_hbm, dstb, ones, zbuf, dacc):
    cid = lax.axis_index("c")
    sid = lax.axis_index("s")

    _zero_fill(zbuf, 64)
    o = jnp.ones((16,), jnp.float32)

    @pl.loop(0, CHUNK)
    def _(r):
        ones[r, pl.ds(0, 16)] = o

    @pl.loop(0, RPT // 64)
    def _(k):
        pltpu.sync_copy(zbuf, dacc.at[pl.ds(sid * RPT + k * 64, 64)])

    row0 = (cid * NS + sid) * _DEG_CHUNKS
    pltpu.sync_copy(dst_hbm.at[pl.ds(row0, _DEG_CHUNKS)], dstb)

    plsc.subcore_barrier()

    @pl.loop(0, _DEG_CHUNKS)
    def _(ci):
        pltpu.sync_copy(ones, dacc.at[dstb.at[ci, 0]], add=True)

    plsc.subcore_barrier()
    pltpu.sync_copy(
        dacc.at[pl.ds(sid * RPT, RPT)], out_hbm.at[cid, pl.ds(sid * RPT, RPT)]
    )


_AGG_CHUNKS = E_PAD // (NS * CHUNK)
_AGG_BATCH = 20
_AGG_NB = _AGG_CHUNKS // _AGG_BATCH


@functools.partial(
    pl.kernel,
    mesh=_MESH,
    out_type=jax.ShapeDtypeStruct((NC, NPAD, DH), jnp.float32),
    scratch_types=[
        pltpu.VMEM((_AGG_BATCH * CHUNK,), jnp.int32),
        pltpu.VMEM((_AGG_BATCH, 1, CHUNK), jnp.int32),
        pltpu.VMEM((CHUNK, DH), jnp.float32),
        pltpu.VMEM((CHUNK, DH), jnp.float32),
        pltpu.VMEM_SHARED((NPAD, DH), jnp.float32),
        pltpu.SemaphoreType.DMA,
        pltpu.SemaphoreType.DMA,
    ],
)
def _agg_kernel(hp_hbm, src_hbm, dst_hbm, out_hbm,
                srcb, dstb, rows0, rows1, acc, sem0, sem1):
    cid = lax.axis_index("c")
    sid = lax.axis_index("s")

    _zero_fill(rows0, 64)

    @pl.loop(0, RPT // 64)
    def _(k):
        pltpu.sync_copy(rows0.at[pl.ds(0, 64)],
                        acc.at[pl.ds(sid * RPT + k * 64, 64)])

    plsc.subcore_barrier()

    off = cid * NPAD

    @pl.loop(0, _AGG_NB)
    def _(b):
        e0 = (sid * _AGG_CHUNKS + b * _AGG_BATCH) * CHUNK
        pltpu.sync_copy(src_hbm.at[pl.ds(e0, _AGG_BATCH * CHUNK)], srcb)
        pltpu.sync_copy(
            dst_hbm.at[pl.ds(sid * _AGG_CHUNKS + b * _AGG_BATCH, _AGG_BATCH)],
            dstb)

        @pl.loop(0, _AGG_BATCH * CHUNK // 16)
        def _(j):
            srcb[pl.ds(j * 16, 16)] = srcb[pl.ds(j * 16, 16)] + off

        pltpu.async_copy(hp_hbm.at[srcb.at[pl.ds(0, CHUNK)]], rows0, sem0)

        @pl.loop(0, _AGG_BATCH, step=2)
        def _(ci):
            pltpu.async_copy(
                hp_hbm.at[srcb.at[pl.ds((ci + 1) * CHUNK, CHUNK)]],
                rows1, sem1)
            pltpu.make_async_copy(
                hp_hbm.at[srcb.at[pl.ds(0, CHUNK)]], rows0, sem0).wait()
            pltpu.sync_copy(rows0, acc.at[dstb.at[ci, 0]], add=True)

            @pl.when(ci + 2 < _AGG_BATCH)
            def _():
                pltpu.async_copy(
                    hp_hbm.at[srcb.at[pl.ds((ci + 2) * CHUNK, CHUNK)]],
                    rows0, sem0)

            pltpu.make_async_copy(
                hp_hbm.at[srcb.at[pl.ds(0, CHUNK)]], rows1, sem1).wait()
            pltpu.sync_copy(rows1, acc.at[dstb.at[ci + 1, 0]], add=True)

    plsc.subcore_barrier()
    pltpu.sync_copy(
        acc.at[pl.ds(sid * RPT, RPT)], out_hbm.at[cid, pl.ds(sid * RPT, RPT)]
    )


def _mm_body(x_ref, w_ref, o_ref):
    o_ref[0] = jnp.dot(x_ref[...], w_ref[...],
                       preferred_element_type=jnp.float32)


def _matmul_stacked(x, w):
    return pl.pallas_call(
        _mm_body,
        grid=(GRID_R, NC),
        in_specs=[
            pl.BlockSpec((ROWB, D), lambda i, j: (i, 0)),
            pl.BlockSpec((D, DH), lambda i, j: (0, j)),
        ],
        out_specs=pl.BlockSpec((1, ROWB, DH), lambda i, j: (j, i, 0)),
        out_shape=jax.ShapeDtypeStruct((NC, NPAD, DH), jnp.float32),
    )(x, w)


def _dinv_of(p0_ref, p1_ref):
    deg = p0_ref[0, :, :1] + p1_ref[0, :, :1] + 1.0
    return lax.rsqrt(deg)


def _scale_body(h_ref, p0_ref, p1_ref, o_ref):
    o_ref[0] = h_ref[0] * _dinv_of(p0_ref, p1_ref)


def _scale_stacked(h, degp):
    return pl.pallas_call(
        _scale_body,
        grid=(GRID_R, NC),
        in_specs=[
            pl.BlockSpec((1, ROWB, DH), lambda i, j: (j, i, 0)),
            pl.BlockSpec((1, ROWB, 16), lambda i, j: (0, i, 0)),
            pl.BlockSpec((1, ROWB, 16), lambda i, j: (1, i, 0)),
        ],
        out_specs=pl.BlockSpec((1, ROWB, DH), lambda i, j: (j, i, 0)),
        out_shape=jax.ShapeDtypeStruct((NC, NPAD, DH), jnp.float32),
    )(h, degp, degp)


def _layer2_body(a0_ref, a1_ref, h0_ref, h1_ref, p0_ref, p1_ref,
                 b_ref, w_ref, o_ref):
    dinv = _dinv_of(p0_ref, p1_ref)
    z0 = jnp.maximum(dinv * (a0_ref[0] + h0_ref[0]) + b_ref[:, :DH], 0.0)
    z1 = jnp.maximum(dinv * (a1_ref[0] + h1_ref[0]) + b_ref[:, DH:], 0.0)
    z = jnp.concatenate([z0, z1], axis=1)
    o_ref[0] = dinv * jnp.dot(z, w_ref[...],
                              preferred_element_type=jnp.float32)


def _layer2(acc1, hp1, degp, b1, w2):
    half = pl.BlockSpec((1, ROWB, DH), lambda i, j: (0, i, 0))
    half1 = pl.BlockSpec((1, ROWB, DH), lambda i, j: (1, i, 0))
    return pl.pallas_call(
        _layer2_body,
        grid=(GRID_R, NC),
        in_specs=[
            half, half1, half, half1,
            pl.BlockSpec((1, ROWB, 16), lambda i, j: (0, i, 0)),
            pl.BlockSpec((1, ROWB, 16), lambda i, j: (1, i, 0)),
            pl.BlockSpec((1, D), lambda i, j: (0, 0)),
            pl.BlockSpec((D, DH), lambda i, j: (0, j)),
        ],
        out_specs=pl.BlockSpec((1, ROWB, DH), lambda i, j: (j, i, 0)),
        out_shape=jax.ShapeDtypeStruct((NC, NPAD, DH), jnp.float32),
    )(acc1, acc1, hp1, hp1, degp, degp, b1, w2)


def _final_body(a_ref, h_ref, p0_ref, p1_ref, b_ref, o_ref):
    dinv = _dinv_of(p0_ref, p1_ref)
    o_ref[...] = dinv * (a_ref[0] + h_ref[0]) + b_ref[0]


def _final(acc2, hp2, degp, b2):
    return pl.pallas_call(
        _final_body,
        grid=(GRID_R, NC),
        in_specs=[
            pl.BlockSpec((1, ROWB, DH), lambda i, j: (j, i, 0)),
            pl.BlockSpec((1, ROWB, DH), lambda i, j: (j, i, 0)),
            pl.BlockSpec((1, ROWB, 16), lambda i, j: (0, i, 0)),
            pl.BlockSpec((1, ROWB, 16), lambda i, j: (1, i, 0)),
            pl.BlockSpec((1, 1, DH), lambda i, j: (j, 0, 0)),
        ],
        out_specs=pl.BlockSpec((ROWB, DH), lambda i, j: (i, j)),
        out_shape=jax.ShapeDtypeStruct((N, D), jnp.float32),
    )(acc2, hp2, degp, degp, b2)


def kernel(x, edge_index, W1, b1, W2, b2):
    src = edge_index[0].astype(jnp.int32)
    dst = edge_index[1].astype(jnp.int32)
    padi = jnp.arange(E_PAD - E, dtype=jnp.int32)
    src1d = jnp.concatenate([src, padi % N])
    dst3d = jnp.concatenate([dst, N + (padi % 64)]).reshape(
        E_PAD // CHUNK, 1, CHUNK)
    b1r = b1.reshape(1, D)
    b2r = b2.reshape(NC, 1, DH)

    h1 = _matmul_stacked(x, W1)
    degp = _deg_kernel(dst3d)
    hp1 = _scale_stacked(h1, degp)
    acc1 = _agg_kernel(hp1.reshape(NC * NPAD, DH), src1d, dst3d)
    hp2 = _layer2(acc1, hp1, degp, b1r, W2)
    acc2 = _agg_kernel(hp2.reshape(NC * NPAD, DH), src1d, dst3d)
    return _final(acc2, hp2, degp, b2r)


# speedup vs baseline: 18.0939x; 1.2575x over previous
import functools

import jax
import jax.numpy as jnp
from jax import lax
from jax.experimental import pallas as pl
from jax.experimental.pallas import tpu as pltpu
from jax.experimental.pallas import tpu_sc as plsc

N = 10000
D = 256
DH = 128
E = 160000
NC = 2
NS = 16
CHUNK = 128
E_PAD = 163840
NPAD = 10240
RPT = NPAD // NS
ROWB = 2000
GRID_R = N // ROWB

_MESH = plsc.VectorSubcoreMesh(core_axis_name="c", subcore_axis_name="s")


def _zero_fill(buf, rows):
    width = buf.shape[1]
    z = jnp.zeros((16,), jnp.float32)

    @pl.loop(0, rows)
    def _(r):
        @pl.loop(0, width // 16)
        def _(c):
            buf[r, pl.ds(c * 16, 16)] = z


_DEG_CHUNKS = E_PAD // (NC * NS * CHUNK)


@functools.partial(
    pl.kernel,
    mesh=_MESH,
    out_type=jax.ShapeDtypeStruct((NC, NPAD, 16), jnp.float32),
    scratch_types=[
        pltpu.VMEM((_DEG_CHUNKS, 1, CHUNK), jnp.int32),
        pltpu.VMEM((CHUNK, 16), jnp.float32),
        pltpu.VMEM((64, 16), jnp.float32),
        pltpu.VMEM_SHARED((NPAD, 16), jnp.float32),
    ],
)
def _deg_kernel(dst_hbm, out_hbm, dstb, ones, zbuf, dacc):
    cid = lax.axis_index("c")
    sid = lax.axis_index("s")

    _zero_fill(zbuf, 64)
    o = jnp.ones((16,), jnp.float32)

    @pl.loop(0, CHUNK)
    def _(r):
        ones[r, pl.ds(0, 16)] = o

    @pl.loop(0, RPT // 64)
    def _(k):
        pltpu.sync_copy(zbuf, dacc.at[pl.ds(sid * RPT + k * 64, 64)])

    row0 = (cid * NS + sid) * _DEG_CHUNKS
    pltpu.sync_copy(dst_hbm.at[pl.ds(row0, _DEG_CHUNKS)], dstb)

    plsc.subcore_barrier()

    @pl.loop(0, _DEG_CHUNKS)
    def _(ci):
        pltpu.sync_copy(ones, dacc.at[dstb.at[ci, 0]], add=True)

    plsc.subcore_barrier()
    pltpu.sync_copy(
        dacc.at[pl.ds(sid * RPT, RPT)], out_hbm.at[cid, pl.ds(sid * RPT, RPT)]
    )


_AGG_CHUNKS = E_PAD // (NS * CHUNK)
_AGG_BATCH = 20
_AGG_NB = _AGG_CHUNKS // _AGG_BATCH


@functools.partial(
    pl.kernel,
    mesh=_MESH,
    out_type=jax.ShapeDtypeStruct((NC, NPAD, DH), jnp.float32),
    scratch_types=[
        pltpu.VMEM((_AGG_BATCH * CHUNK,), jnp.int32),
        pltpu.VMEM((_AGG_BATCH, 1, CHUNK), jnp.int32),
        pltpu.VMEM((CHUNK, DH), jnp.float32),
        pltpu.VMEM((CHUNK, DH), jnp.float32),
        pltpu.VMEM_SHARED((NPAD, DH), jnp.float32),
        pltpu.SemaphoreType.DMA,
        pltpu.SemaphoreType.DMA,
    ],
)
def _agg_kernel(hp_hbm, src_hbm, dst_hbm, out_hbm,
                srcb, dstb, rows0, rows1, acc, sem0, sem1):
    cid = lax.axis_index("c")
    sid = lax.axis_index("s")

    _zero_fill(rows0, 64)

    @pl.loop(0, RPT // 64)
    def _(k):
        pltpu.sync_copy(rows0.at[pl.ds(0, 64)],
                        acc.at[pl.ds(sid * RPT + k * 64, 64)])

    plsc.subcore_barrier()

    off = cid * NPAD

    @pl.loop(0, _AGG_NB)
    def _(b):
        e0 = (sid * _AGG_CHUNKS + b * _AGG_BATCH) * CHUNK
        pltpu.sync_copy(src_hbm.at[pl.ds(e0, _AGG_BATCH * CHUNK)], srcb)
        pltpu.sync_copy(
            dst_hbm.at[pl.ds(sid * _AGG_CHUNKS + b * _AGG_BATCH, _AGG_BATCH)],
            dstb)

        @pl.loop(0, _AGG_BATCH * CHUNK // 16)
        def _(j):
            srcb[pl.ds(j * 16, 16)] = srcb[pl.ds(j * 16, 16)] + off

        pltpu.async_copy(hp_hbm.at[srcb.at[pl.ds(0, CHUNK)]], rows0, sem0)

        @pl.loop(0, _AGG_BATCH, step=2)
        def _(ci):
            pltpu.async_copy(
                hp_hbm.at[srcb.at[pl.ds((ci + 1) * CHUNK, CHUNK)]],
                rows1, sem1)
            pltpu.make_async_copy(
                hp_hbm.at[srcb.at[pl.ds(0, CHUNK)]], rows0, sem0).wait()
            pltpu.sync_copy(rows0, acc.at[dstb.at[ci, 0]], add=True)

            @pl.when(ci + 2 < _AGG_BATCH)
            def _():
                pltpu.async_copy(
                    hp_hbm.at[srcb.at[pl.ds((ci + 2) * CHUNK, CHUNK)]],
                    rows0, sem0)

            pltpu.make_async_copy(
                hp_hbm.at[srcb.at[pl.ds(0, CHUNK)]], rows1, sem1).wait()
            pltpu.sync_copy(rows1, acc.at[dstb.at[ci + 1, 0]], add=True)

    plsc.subcore_barrier()
    pltpu.sync_copy(
        acc.at[pl.ds(sid * RPT, RPT)], out_hbm.at[cid, pl.ds(sid * RPT, RPT)]
    )


def _mm_body(x_ref, w_ref, o_ref):
    o_ref[0] = jnp.dot(x_ref[...], w_ref[...],
                       preferred_element_type=jnp.float32)


def _matmul_stacked(x, w):
    return pl.pallas_call(
        _mm_body,
        grid=(GRID_R, NC),
        in_specs=[
            pl.BlockSpec((ROWB, D), lambda i, j: (i, 0)),
            pl.BlockSpec((D, DH), lambda i, j: (0, j)),
        ],
        out_specs=pl.BlockSpec((1, ROWB, DH), lambda i, j: (j, i, 0)),
        out_shape=jax.ShapeDtypeStruct((NC, NPAD, DH), jnp.float32),
    )(x, w)


def _dinv_of(p0_ref, p1_ref):
    deg = p0_ref[0, :, :1] + p1_ref[0, :, :1] + 1.0
    return lax.rsqrt(deg)


def _scale_body(h_ref, p0_ref, p1_ref, o_ref):
    o_ref[0] = h_ref[0] * _dinv_of(p0_ref, p1_ref)


def _scale_stacked(h, degp):
    return pl.pallas_call(
        _scale_body,
        grid=(GRID_R, NC),
        in_specs=[
            pl.BlockSpec((1, ROWB, DH), lambda i, j: (j, i, 0)),
            pl.BlockSpec((1, ROWB, 16), lambda i, j: (0, i, 0)),
            pl.BlockSpec((1, ROWB, 16), lambda i, j: (1, i, 0)),
        ],
        out_specs=pl.BlockSpec((1, ROWB, DH), lambda i, j: (j, i, 0)),
        out_shape=jax.ShapeDtypeStruct((NC, NPAD, DH), jnp.float32),
    )(h, degp, degp)


def _layer2_body(a0_ref, a1_ref, h0_ref, h1_ref, p0_ref, p1_ref,
                 b_ref, w_ref, o_ref):
    dinv = _dinv_of(p0_ref, p1_ref)
    z0 = jnp.maximum(dinv * (a0_ref[0] + h0_ref[0]) + b_ref[:, :DH], 0.0)
    z1 = jnp.maximum(dinv * (a1_ref[0] + h1_ref[0]) + b_ref[:, DH:], 0.0)
    z = jnp.concatenate([z0, z1], axis=1)
    o_ref[0] = dinv * jnp.dot(z, w_ref[...],
                              preferred_element_type=jnp.float32)


def _layer2(acc1, hp1, degp, b1, w2):
    half = pl.BlockSpec((1, ROWB, DH), lambda i, j: (0, i, 0))
    half1 = pl.BlockSpec((1, ROWB, DH), lambda i, j: (1, i, 0))
    return pl.pallas_call(
        _layer2_body,
        grid=(GRID_R, NC),
        in_specs=[
            half, half1, half, half1,
            pl.BlockSpec((1, ROWB, 16), lambda i, j: (0, i, 0)),
            pl.BlockSpec((1, ROWB, 16), lambda i, j: (1, i, 0)),
            pl.BlockSpec((1, D), lambda i, j: (0, 0)),
            pl.BlockSpec((D, DH), lambda i, j: (0, j)),
        ],
        out_specs=pl.BlockSpec((1, ROWB, DH), lambda i, j: (j, i, 0)),
        out_shape=jax.ShapeDtypeStruct((NC, NPAD, DH), jnp.float32),
    )(acc1, acc1, hp1, hp1, degp, degp, b1, w2)


def _final_body(a_ref, h_ref, p0_ref, p1_ref, b_ref, o_ref):
    dinv = _dinv_of(p0_ref, p1_ref)
    o_ref[...] = dinv * (a_ref[0] + h_ref[0]) + b_ref[0]


def _final(acc2, hp2, degp, b2):
    return pl.pallas_call(
        _final_body,
        grid=(GRID_R, NC),
        in_specs=[
            pl.BlockSpec((1, ROWB, DH), lambda i, j: (j, i, 0)),
            pl.BlockSpec((1, ROWB, DH), lambda i, j: (j, i, 0)),
            pl.BlockSpec((1, ROWB, 16), lambda i, j: (0, i, 0)),
            pl.BlockSpec((1, ROWB, 16), lambda i, j: (1, i, 0)),
            pl.BlockSpec((1, 1, DH), lambda i, j: (j, 0, 0)),
        ],
        out_specs=pl.BlockSpec((ROWB, DH), lambda i, j: (i, j)),
        out_shape=jax.ShapeDtypeStruct((N, D), jnp.float32),
    )(acc2, hp2, degp, degp, b2)


def kernel(x, edge_index, W1, b1, W2, b2):
    src = edge_index[0].astype(jnp.int32)
    dst = edge_index[1].astype(jnp.int32)
    padi = jnp.arange(E_PAD - E, dtype=jnp.int32)
    src1d = jnp.concatenate([src, padi % N])
    dst3d = jnp.concatenate([dst, N + (padi % 64)]).reshape(
        E_PAD // CHUNK, 1, CHUNK)
    b1r = b1.reshape(1, D)
    b2r = b2.reshape(NC, 1, DH)

    h1 = _matmul_stacked(x, W1)
    degp = _deg_kernel(dst3d)
    hp1 = _scale_stacked(h1, degp)
    acc1 = _agg_kernel(hp1.reshape(NC * NPAD, DH), src1d, dst3d)
    hp2 = _layer2(acc1, hp1, degp, b1r, W2)
    acc2 = _agg_kernel(hp2.reshape(NC * NPAD, DH), src1d, dst3d)
    return _final(acc2, hp2, degp, b2r)


# speedup vs baseline: 18.3243x; 1.0127x over previous
import functools

import jax
import jax.numpy as jnp
from jax import lax
from jax.experimental import pallas as pl
from jax.experimental.pallas import tpu as pltpu
from jax.experimental.pallas import tpu_sc as plsc

N = 10000
D = 256
DH = 128
E = 160000
NC = 2
NS = 16
CHUNK = 128
NCHUNK = E // CHUNK
NPAD = 10240
RPT = NPAD // NS
ROWB = 2000
GRID_R = N // ROWB

_MESH = plsc.VectorSubcoreMesh(core_axis_name="c", subcore_axis_name="s")


def _zero_fill(buf, rows):
    width = buf.shape[1]
    if buf.dtype == jnp.bfloat16:
        z = jnp.zeros((2, 16), jnp.bfloat16)

        @pl.loop(0, rows // 2)
        def _(r):
            r2 = pl.multiple_of(r * 2, 2)

            @pl.loop(0, width // 16)
            def _(c):
                buf[pl.ds(r2, 2), pl.ds(c * 16, 16)] = z
    else:
        z = jnp.zeros((16,), buf.dtype)

        @pl.loop(0, rows)
        def _(r):
            @pl.loop(0, width // 16)
            def _(c):
                buf[r, pl.ds(c * 16, 16)] = z


_DEG_CHUNKS = NCHUNK // (NC * NS)
_DEG_EXTRA = NCHUNK - _DEG_CHUNKS * NC * NS


@functools.partial(
    pl.kernel,
    mesh=_MESH,
    out_type=jax.ShapeDtypeStruct((NC, NPAD, 16), jnp.float32),
    scratch_types=[
        pltpu.VMEM((_DEG_CHUNKS, 1, CHUNK), jnp.int32),
        pltpu.VMEM((CHUNK, 16), jnp.float32),
        pltpu.VMEM((64, 16), jnp.float32),
        pltpu.VMEM_SHARED((NPAD, 16), jnp.float32),
    ],
)
def _deg_kernel(dst_hbm, out_hbm, dstb, ones, zbuf, dacc):
    cid = lax.axis_index("c")
    sid = lax.axis_index("s")

    _zero_fill(zbuf, 64)
    o = jnp.ones((16,), jnp.float32)

    @pl.loop(0, CHUNK)
    def _(r):
        ones[r, pl.ds(0, 16)] = o

    @pl.loop(0, RPT // 64)
    def _(k):
        pltpu.sync_copy(zbuf, dacc.at[pl.ds(sid * RPT + k * 64, 64)])

    wid = cid * NS + sid
    row0 = wid * _DEG_CHUNKS
    pltpu.sync_copy(dst_hbm.at[pl.ds(row0, _DEG_CHUNKS)], dstb)

    plsc.subcore_barrier()

    @pl.loop(0, _DEG_CHUNKS)
    def _(ci):
        pltpu.sync_copy(ones, dacc.at[dstb.at[ci, 0]], add=True)

    @pl.when(wid < _DEG_EXTRA)
    def _():
        pltpu.sync_copy(
            dst_hbm.at[pl.ds(NC * NS * _DEG_CHUNKS + wid, 1)],
            dstb.at[pl.ds(0, 1)])
        pltpu.sync_copy(ones, dacc.at[dstb.at[0, 0]], add=True)

    plsc.subcore_barrier()
    pltpu.sync_copy(
        dacc.at[pl.ds(sid * RPT, RPT)], out_hbm.at[cid, pl.ds(sid * RPT, RPT)]
    )


_AGG_CHUNKS = NCHUNK // NS
_AGG_BATCH = 26
_AGG_NB = _AGG_CHUNKS // _AGG_BATCH
_AGG_EXTRA = NCHUNK - _AGG_CHUNKS * NS


@functools.partial(
    pl.kernel,
    mesh=_MESH,
    out_type=jax.ShapeDtypeStruct((NC, NPAD, DH), jnp.float32),
    scratch_types=[
        pltpu.VMEM((_AGG_BATCH * CHUNK,), jnp.int32),
        pltpu.VMEM((_AGG_BATCH, 1, CHUNK), jnp.int32),
        pltpu.VMEM((CHUNK, DH), jnp.float32),
        pltpu.VMEM((CHUNK, DH), jnp.float32),
        pltpu.VMEM_SHARED((NPAD, DH), jnp.float32),
        pltpu.SemaphoreType.DMA,
        pltpu.SemaphoreType.DMA,
    ],
)
def _agg_kernel(hp_hbm, src_hbm, dst_hbm, out_hbm,
                srcb, dstb, rows0, rows1, acc, sem0, sem1):
    cid = lax.axis_index("c")
    sid = lax.axis_index("s")

    _zero_fill(rows0, 64)

    @pl.loop(0, RPT // 64)
    def _(k):
        pltpu.sync_copy(rows0.at[pl.ds(0, 64)],
                        acc.at[pl.ds(sid * RPT + k * 64, 64)])

    plsc.subcore_barrier()

    off = cid * NPAD

    @pl.loop(0, _AGG_NB)
    def _(b):
        e0 = (sid * _AGG_CHUNKS + b * _AGG_BATCH) * CHUNK
        pltpu.sync_copy(src_hbm.at[pl.ds(e0, _AGG_BATCH * CHUNK)], srcb)
        pltpu.sync_copy(
            dst_hbm.at[pl.ds(sid * _AGG_CHUNKS + b * _AGG_BATCH, _AGG_BATCH)],
            dstb)

        @pl.loop(0, _AGG_BATCH * CHUNK // 16)
        def _(j):
            srcb[pl.ds(j * 16, 16)] = srcb[pl.ds(j * 16, 16)] + off

        pltpu.async_copy(hp_hbm.at[srcb.at[pl.ds(0, CHUNK)]], rows0, sem0)

        @pl.loop(0, _AGG_BATCH, step=2)
        def _(ci):
            pltpu.async_copy(
                hp_hbm.at[srcb.at[pl.ds((ci + 1) * CHUNK, CHUNK)]],
                rows1, sem1)
            pltpu.make_async_copy(
                hp_hbm.at[srcb.at[pl.ds(0, CHUNK)]], rows0, sem0).wait()
            pltpu.sync_copy(rows0, acc.at[dstb.at[ci, 0]], add=True)

            @pl.when(ci + 2 < _AGG_BATCH)
            def _():
                pltpu.async_copy(
                    hp_hbm.at[srcb.at[pl.ds((ci + 2) * CHUNK, CHUNK)]],
                    rows0, sem0)

            pltpu.make_async_copy(
                hp_hbm.at[srcb.at[pl.ds(0, CHUNK)]], rows1, sem1).wait()
            pltpu.sync_copy(rows1, acc.at[dstb.at[ci + 1, 0]], add=True)

    @pl.when(sid < _AGG_EXTRA)
    def _():
        ce = NS * _AGG_CHUNKS + sid
        pltpu.sync_copy(src_hbm.at[pl.ds(ce * CHUNK, CHUNK)],
                        srcb.at[pl.ds(0, CHUNK)])
        pltpu.sync_copy(dst_hbm.at[pl.ds(ce, 1)], dstb.at[pl.ds(0, 1)])

        @pl.loop(0, CHUNK // 16)
        def _(j):
            srcb[pl.ds(j * 16, 16)] = srcb[pl.ds(j * 16, 16)] + off

        pltpu.async_copy(hp_hbm.at[srcb.at[pl.ds(0, CHUNK)]],
                         rows0, sem0).wait()
        pltpu.sync_copy(rows0, acc.at[dstb.at[0, 0]], add=True)

    plsc.subcore_barrier()
    pltpu.sync_copy(
        acc.at[pl.ds(sid * RPT, RPT)], out_hbm.at[cid, pl.ds(sid * RPT, RPT)]
    )


def _mm_body(x_ref, w_ref, o_ref):
    o_ref[0] = jnp.dot(x_ref[...], w_ref[...],
                       preferred_element_type=jnp.float32)


def _matmul_stacked(x, w):
    return pl.pallas_call(
        _mm_body,
        grid=(GRID_R, NC),
        in_specs=[
            pl.BlockSpec((ROWB, D), lambda i, j: (i, 0)),
            pl.BlockSpec((D, DH), lambda i, j: (0, j)),
        ],
        out_specs=pl.BlockSpec((1, ROWB, DH), lambda i, j: (j, i, 0)),
        out_shape=jax.ShapeDtypeStruct((NC, NPAD, DH), jnp.float32),
    )(x, w)


def _dinv_of(p0_ref, p1_ref):
    deg = p0_ref[0, :, :1] + p1_ref[0, :, :1] + 1.0
    return lax.rsqrt(deg)


def _scale_body(h_ref, p0_ref, p1_ref, o_ref):
    o_ref[0] = h_ref[0] * _dinv_of(p0_ref, p1_ref)


def _scale_stacked(h, degp):
    return pl.pallas_call(
        _scale_body,
        grid=(GRID_R, NC),
        in_specs=[
            pl.BlockSpec((1, ROWB, DH), lambda i, j: (j, i, 0)),
            pl.BlockSpec((1, ROWB, 16), lambda i, j: (0, i, 0)),
            pl.BlockSpec((1, ROWB, 16), lambda i, j: (1, i, 0)),
        ],
        out_specs=pl.BlockSpec((1, ROWB, DH), lambda i, j: (j, i, 0)),
        out_shape=jax.ShapeDtypeStruct((NC, NPAD, DH), jnp.float32),
    )(h, degp, degp)


def _layer2_body(a0_ref, a1_ref, h0_ref, h1_ref, p0_ref, p1_ref,
                 b_ref, w_ref, o_ref):
    dinv = _dinv_of(p0_ref, p1_ref)
    a0 = a0_ref[0].astype(jnp.float32) + h0_ref[0].astype(jnp.float32)
    a1 = a1_ref[0].astype(jnp.float32) + h1_ref[0].astype(jnp.float32)
    z0 = jnp.maximum(dinv * a0 + b_ref[:, :DH], 0.0)
    z1 = jnp.maximum(dinv * a1 + b_ref[:, DH:], 0.0)
    z = jnp.concatenate([z0, z1], axis=1)
    o_ref[0] = dinv * jnp.dot(z, w_ref[...],
                              preferred_element_type=jnp.float32)


def _layer2(acc1, hp1, degp, b1, w2):
    half = pl.BlockSpec((1, ROWB, DH), lambda i, j: (0, i, 0))
    half1 = pl.BlockSpec((1, ROWB, DH), lambda i, j: (1, i, 0))
    return pl.pallas_call(
        _layer2_body,
        grid=(GRID_R, NC),
        in_specs=[
            half, half1, half, half1,
            pl.BlockSpec((1, ROWB, 16), lambda i, j: (0, i, 0)),
            pl.BlockSpec((1, ROWB, 16), lambda i, j: (1, i, 0)),
            pl.BlockSpec((1, D), lambda i, j: (0, 0)),
            pl.BlockSpec((D, DH), lambda i, j: (0, j)),
        ],
        out_specs=pl.BlockSpec((1, ROWB, DH), lambda i, j: (j, i, 0)),
        out_shape=jax.ShapeDtypeStruct((NC, NPAD, DH), jnp.float32),
    )(acc1, acc1, hp1, hp1, degp, degp, b1, w2)


def _final_body(a_ref, h_ref, p0_ref, p1_ref, b_ref, o_ref):
    dinv = _dinv_of(p0_ref, p1_ref)
    s = a_ref[0].astype(jnp.float32) + h_ref[0].astype(jnp.float32)
    o_ref[...] = dinv * s + b_ref[0]


def _final(acc2, hp2, degp, b2):
    return pl.pallas_call(
        _final_body,
        grid=(GRID_R, NC),
        in_specs=[
            pl.BlockSpec((1, ROWB, DH), lambda i, j: (j, i, 0)),
            pl.BlockSpec((1, ROWB, DH), lambda i, j: (j, i, 0)),
            pl.BlockSpec((1, ROWB, 16), lambda i, j: (0, i, 0)),
            pl.BlockSpec((1, ROWB, 16), lambda i, j: (1, i, 0)),
            pl.BlockSpec((1, 1, DH), lambda i, j: (j, 0, 0)),
        ],
        out_specs=pl.BlockSpec((ROWB, DH), lambda i, j: (i, j)),
        out_shape=jax.ShapeDtypeStruct((N, D), jnp.float32),
    )(acc2, hp2, degp, degp, b2)


def kernel(x, edge_index, W1, b1, W2, b2):
    src1d = edge_index[0].astype(jnp.int32)
    dst3d = edge_index[1].astype(jnp.int32).reshape(NCHUNK, 1, CHUNK)
    b1r = b1.reshape(1, D)
    b2r = b2.reshape(NC, 1, DH)

    h1 = _matmul_stacked(x, W1)
    degp = _deg_kernel(dst3d)
    hp1 = _scale_stacked(h1, degp)
    acc1 = _agg_kernel(hp1.reshape(NC * NPAD, DH), src1d, dst3d)
    hp2 = _layer2(acc1, hp1, degp, b1r, W2)
    acc2 = _agg_kernel(hp2.reshape(NC * NPAD, DH), src1d, dst3d)
    return _final(acc2, hp2, degp, b2r)


# speedup vs baseline: 18.4753x; 1.0082x over previous
import functools

import jax
import jax.numpy as jnp
from jax import lax
from jax.experimental import pallas as pl
from jax.experimental.pallas import tpu as pltpu
from jax.experimental.pallas import tpu_sc as plsc

N = 10000
D = 256
DH = 128
E = 160000
NC = 2
NS = 16
CHUNK = 128
NCHUNK = E // CHUNK
NPAD = 10240
RPT = NPAD // NS
ROWB = 2000
GRID_R = N // ROWB

_MESH = plsc.VectorSubcoreMesh(core_axis_name="c", subcore_axis_name="s")


def _zero_fill(buf, rows):
    width = buf.shape[1]
    if buf.dtype == jnp.bfloat16:
        z = jnp.zeros((2, 16), jnp.bfloat16)

        @pl.loop(0, rows // 2)
        def _(r):
            r2 = pl.multiple_of(r * 2, 2)

            @pl.loop(0, width // 16)
            def _(c):
                buf[pl.ds(r2, 2), pl.ds(c * 16, 16)] = z
    else:
        z = jnp.zeros((16,), buf.dtype)

        @pl.loop(0, rows)
        def _(r):
            @pl.loop(0, width // 16)
            def _(c):
                buf[r, pl.ds(c * 16, 16)] = z


_DEG_CHUNKS = NCHUNK // (NC * NS)
_DEG_EXTRA = NCHUNK - _DEG_CHUNKS * NC * NS


@functools.partial(
    pl.kernel,
    mesh=_MESH,
    out_type=jax.ShapeDtypeStruct((NC, NPAD, 16), jnp.float32),
    scratch_types=[
        pltpu.VMEM((_DEG_CHUNKS, 1, CHUNK), jnp.int32),
        pltpu.VMEM((CHUNK, 16), jnp.float32),
        pltpu.VMEM((64, 16), jnp.float32),
        pltpu.VMEM_SHARED((NPAD, 16), jnp.float32),
    ],
)
def _deg_kernel(dst_hbm, out_hbm, dstb, ones, zbuf, dacc):
    cid = lax.axis_index("c")
    sid = lax.axis_index("s")

    _zero_fill(zbuf, 64)
    o = jnp.ones((16,), jnp.float32)

    @pl.loop(0, CHUNK)
    def _(r):
        ones[r, pl.ds(0, 16)] = o

    @pl.loop(0, RPT // 64)
    def _(k):
        pltpu.sync_copy(zbuf, dacc.at[pl.ds(sid * RPT + k * 64, 64)])

    wid = cid * NS + sid
    row0 = NCHUNK + wid * _DEG_CHUNKS
    pltpu.sync_copy(dst_hbm.at[pl.ds(row0, _DEG_CHUNKS)], dstb)

    plsc.subcore_barrier()

    @pl.loop(0, _DEG_CHUNKS)
    def _(ci):
        pltpu.sync_copy(ones, dacc.at[dstb.at[ci, 0]], add=True)

    @pl.when(wid < _DEG_EXTRA)
    def _():
        pltpu.sync_copy(
            dst_hbm.at[pl.ds(NCHUNK + NC * NS * _DEG_CHUNKS + wid, 1)],
            dstb.at[pl.ds(0, 1)])
        pltpu.sync_copy(ones, dacc.at[dstb.at[0, 0]], add=True)

    plsc.subcore_barrier()
    pltpu.sync_copy(
        dacc.at[pl.ds(sid * RPT, RPT)], out_hbm.at[cid, pl.ds(sid * RPT, RPT)]
    )


_AGG_CHUNKS = NCHUNK // NS
_AGG_BATCH = 26
_AGG_NB = _AGG_CHUNKS // _AGG_BATCH
_AGG_EXTRA = NCHUNK - _AGG_CHUNKS * NS


@functools.partial(
    pl.kernel,
    mesh=_MESH,
    out_type=jax.ShapeDtypeStruct((NC, NPAD, DH), jnp.float32),
    scratch_types=[
        pltpu.VMEM((_AGG_BATCH * CHUNK,), jnp.int32),
        pltpu.VMEM((_AGG_BATCH, 1, CHUNK), jnp.int32),
        pltpu.VMEM((CHUNK, DH), jnp.float32),
        pltpu.VMEM((CHUNK, DH), jnp.float32),
        pltpu.VMEM_SHARED((NPAD, DH), jnp.float32),
        pltpu.SemaphoreType.DMA,
        pltpu.SemaphoreType.DMA,
    ],
)
def _agg_kernel(hp_hbm, src_hbm, dst_hbm, out_hbm,
                srcb, dstb, rows0, rows1, acc, sem0, sem1):
    cid = lax.axis_index("c")
    sid = lax.axis_index("s")

    _zero_fill(rows0, 64)

    @pl.loop(0, RPT // 64)
    def _(k):
        pltpu.sync_copy(rows0.at[pl.ds(0, 64)],
                        acc.at[pl.ds(sid * RPT + k * 64, 64)])

    plsc.subcore_barrier()

    off = cid * NPAD

    @pl.loop(0, _AGG_NB)
    def _(b):
        e0 = (sid * _AGG_CHUNKS + b * _AGG_BATCH) * CHUNK
        pltpu.sync_copy(src_hbm.at[pl.ds(e0, _AGG_BATCH * CHUNK)], srcb)
        pltpu.sync_copy(
            dst_hbm.at[pl.ds(NCHUNK + sid * _AGG_CHUNKS + b * _AGG_BATCH,
                             _AGG_BATCH)],
            dstb)

        @pl.loop(0, _AGG_BATCH * CHUNK // 16)
        def _(j):
            srcb[pl.ds(j * 16, 16)] = srcb[pl.ds(j * 16, 16)] + off

        pltpu.async_copy(hp_hbm.at[srcb.at[pl.ds(0, CHUNK)]], rows0, sem0)

        @pl.loop(0, _AGG_BATCH, step=2)
        def _(ci):
            pltpu.async_copy(
                hp_hbm.at[srcb.at[pl.ds((ci + 1) * CHUNK, CHUNK)]],
                rows1, sem1)
            pltpu.make_async_copy(
                hp_hbm.at[srcb.at[pl.ds(0, CHUNK)]], rows0, sem0).wait()
            pltpu.sync_copy(rows0, acc.at[dstb.at[ci, 0]], add=True)

            @pl.when(ci + 2 < _AGG_BATCH)
            def _():
                pltpu.async_copy(
                    hp_hbm.at[srcb.at[pl.ds((ci + 2) * CHUNK, CHUNK)]],
                    rows0, sem0)

            pltpu.make_async_copy(
                hp_hbm.at[srcb.at[pl.ds(0, CHUNK)]], rows1, sem1).wait()
            pltpu.sync_copy(rows1, acc.at[dstb.at[ci + 1, 0]], add=True)

    @pl.when(sid < _AGG_EXTRA)
    def _():
        ce = NS * _AGG_CHUNKS + sid
        pltpu.sync_copy(src_hbm.at[pl.ds(ce * CHUNK, CHUNK)],
                        srcb.at[pl.ds(0, CHUNK)])
        pltpu.sync_copy(dst_hbm.at[pl.ds(NCHUNK + ce, 1)],
                        dstb.at[pl.ds(0, 1)])

        @pl.loop(0, CHUNK // 16)
        def _(j):
            srcb[pl.ds(j * 16, 16)] = srcb[pl.ds(j * 16, 16)] + off

        pltpu.async_copy(hp_hbm.at[srcb.at[pl.ds(0, CHUNK)]],
                         rows0, sem0).wait()
        pltpu.sync_copy(rows0, acc.at[dstb.at[0, 0]], add=True)

    plsc.subcore_barrier()
    pltpu.sync_copy(
        acc.at[pl.ds(sid * RPT, RPT)], out_hbm.at[cid, pl.ds(sid * RPT, RPT)]
    )


def _mm_body(x_ref, w_ref, o_ref):
    o_ref[0] = jnp.dot(x_ref[...], w_ref[...],
                       preferred_element_type=jnp.float32)


def _matmul_stacked(x, w):
    return pl.pallas_call(
        _mm_body,
        grid=(GRID_R, NC),
        in_specs=[
            pl.BlockSpec((ROWB, D), lambda i, j: (i, 0)),
            pl.BlockSpec((D, DH), lambda i, j: (0, j)),
        ],
        out_specs=pl.BlockSpec((1, ROWB, DH), lambda i, j: (j, i, 0)),
        out_shape=jax.ShapeDtypeStruct((NC, NPAD, DH), jnp.float32),
    )(x, w)


def _dinv_of(p0_ref, p1_ref):
    deg = p0_ref[0, :, :1] + p1_ref[0, :, :1] + 1.0
    return lax.rsqrt(deg)


def _scale_body(h_ref, p0_ref, p1_ref, o_ref):
    o_ref[0] = h_ref[0] * _dinv_of(p0_ref, p1_ref)


def _scale_stacked(h, degp):
    return pl.pallas_call(
        _scale_body,
        grid=(GRID_R, NC),
        in_specs=[
            pl.BlockSpec((1, ROWB, DH), lambda i, j: (j, i, 0)),
            pl.BlockSpec((1, ROWB, 16), lambda i, j: (0, i, 0)),
            pl.BlockSpec((1, ROWB, 16), lambda i, j: (1, i, 0)),
        ],
        out_specs=pl.BlockSpec((1, ROWB, DH), lambda i, j: (j, i, 0)),
        out_shape=jax.ShapeDtypeStruct((NC, NPAD, DH), jnp.float32),
    )(h, degp, degp)


def _layer2_body(a0_ref, a1_ref, h0_ref, h1_ref, p0_ref, p1_ref,
                 b_ref, w_ref, o_ref):
    dinv = _dinv_of(p0_ref, p1_ref)
    a0 = a0_ref[0].astype(jnp.float32) + h0_ref[0].astype(jnp.float32)
    a1 = a1_ref[0].astype(jnp.float32) + h1_ref[0].astype(jnp.float32)
    z0 = jnp.maximum(dinv * a0 + b_ref[:, :DH], 0.0)
    z1 = jnp.maximum(dinv * a1 + b_ref[:, DH:], 0.0)
    z = jnp.concatenate([z0, z1], axis=1)
    o_ref[0] = dinv * jnp.dot(z, w_ref[...],
                              preferred_element_type=jnp.float32)


def _layer2(acc1, hp1, degp, b1, w2):
    half = pl.BlockSpec((1, ROWB, DH), lambda i, j: (0, i, 0))
    half1 = pl.BlockSpec((1, ROWB, DH), lambda i, j: (1, i, 0))
    return pl.pallas_call(
        _layer2_body,
        grid=(GRID_R, NC),
        in_specs=[
            half, half1, half, half1,
            pl.BlockSpec((1, ROWB, 16), lambda i, j: (0, i, 0)),
            pl.BlockSpec((1, ROWB, 16), lambda i, j: (1, i, 0)),
            pl.BlockSpec((1, D), lambda i, j: (0, 0)),
            pl.BlockSpec((D, DH), lambda i, j: (0, j)),
        ],
        out_specs=pl.BlockSpec((1, ROWB, DH), lambda i, j: (j, i, 0)),
        out_shape=jax.ShapeDtypeStruct((NC, NPAD, DH), jnp.float32),
    )(acc1, acc1, hp1, hp1, degp, degp, b1, w2)


def _final_body(a_ref, h_ref, p0_ref, p1_ref, b_ref, o_ref):
    dinv = _dinv_of(p0_ref, p1_ref)
    s = a_ref[0].astype(jnp.float32) + h_ref[0].astype(jnp.float32)
    o_ref[...] = dinv * s + b_ref[0]


def _final(acc2, hp2, degp, b2):
    return pl.pallas_call(
        _final_body,
        grid=(GRID_R, NC),
        in_specs=[
            pl.BlockSpec((1, ROWB, DH), lambda i, j: (j, i, 0)),
            pl.BlockSpec((1, ROWB, DH), lambda i, j: (j, i, 0)),
            pl.BlockSpec((1, ROWB, 16), lambda i, j: (0, i, 0)),
            pl.BlockSpec((1, ROWB, 16), lambda i, j: (1, i, 0)),
            pl.BlockSpec((1, 1, DH), lambda i, j: (j, 0, 0)),
        ],
        out_specs=pl.BlockSpec((ROWB, DH), lambda i, j: (i, j)),
        out_shape=jax.ShapeDtypeStruct((N, D), jnp.float32),
    )(acc2, hp2, degp, degp, b2)


def kernel(x, edge_index, W1, b1, W2, b2):
    ei = edge_index.astype(jnp.int32)
    src1d = ei.reshape(2 * E)
    dst3d = ei.reshape(2 * NCHUNK, 1, CHUNK)
    b1r = b1.reshape(1, D)
    b2r = b2.reshape(NC, 1, DH)

    h1 = _matmul_stacked(x, W1)
    degp = _deg_kernel(dst3d)
    hp1 = _scale_stacked(h1, degp)
    acc1 = _agg_kernel(hp1.reshape(NC * NPAD, DH), src1d, dst3d)
    hp2 = _layer2(acc1, hp1, degp, b1r, W2)
    acc2 = _agg_kernel(hp2.reshape(NC * NPAD, DH), src1d, dst3d)
    return _final(acc2, hp2, degp, b2r)


# speedup vs baseline: 18.6417x; 1.0090x over previous
import functools

import jax
import jax.numpy as jnp
from jax import lax
from jax.experimental import pallas as pl
from jax.experimental.pallas import tpu as pltpu
from jax.experimental.pallas import tpu_sc as plsc

N = 10000
D = 256
DH = 128
E = 160000
NC = 2
NS = 16
CHUNK = 128
NCHUNK = E // CHUNK
NPAD = 10240
RPT = NPAD // NS
ROWB = 2000
GRID_R = N // ROWB

_MESH = plsc.VectorSubcoreMesh(core_axis_name="c", subcore_axis_name="s")


def _zero_fill(buf, rows):
    width = buf.shape[1]
    if buf.dtype == jnp.bfloat16:
        z = jnp.zeros((2, 16), jnp.bfloat16)

        @pl.loop(0, rows // 2)
        def _(r):
            r2 = pl.multiple_of(r * 2, 2)

            @pl.loop(0, width // 16)
            def _(c):
                buf[pl.ds(r2, 2), pl.ds(c * 16, 16)] = z
    else:
        z = jnp.zeros((16,), buf.dtype)

        @pl.loop(0, rows)
        def _(r):
            @pl.loop(0, width // 16)
            def _(c):
                buf[r, pl.ds(c * 16, 16)] = z


_DEG_CHUNKS = NCHUNK // (NC * NS)
_DEG_EXTRA = NCHUNK - _DEG_CHUNKS * NC * NS


@functools.partial(
    pl.kernel,
    mesh=_MESH,
    out_type=jax.ShapeDtypeStruct((NC, NPAD, 16), jnp.float32),
    scratch_types=[
        pltpu.VMEM((_DEG_CHUNKS, 1, CHUNK), jnp.int32),
        pltpu.VMEM((CHUNK, 16), jnp.float32),
        pltpu.VMEM((64, 16), jnp.float32),
        pltpu.VMEM_SHARED((NPAD, 16), jnp.float32),
        pltpu.SemaphoreType.DMA,
    ],
)
def _deg_kernel(dst_hbm, out_hbm, dstb, ones, zbuf, dacc, zsem):
    cid = lax.axis_index("c")
    sid = lax.axis_index("s")

    _zero_fill(zbuf, 64)
    o = jnp.ones((16,), jnp.float32)

    @pl.loop(0, CHUNK)
    def _(r):
        ones[r, pl.ds(0, 16)] = o

    @pl.loop(0, RPT // 64)
    def _(k):
        pltpu.async_copy(zbuf, dacc.at[pl.ds(sid * RPT + k * 64, 64)], zsem)

    @pl.loop(0, RPT // 64)
    def _(k):
        pltpu.make_async_copy(zbuf, dacc.at[pl.ds(sid * RPT, 64)],
                              zsem).wait()

    wid = cid * NS + sid
    row0 = NCHUNK + wid * _DEG_CHUNKS
    pltpu.sync_copy(dst_hbm.at[pl.ds(row0, _DEG_CHUNKS)], dstb)

    plsc.subcore_barrier()

    @pl.loop(0, _DEG_CHUNKS)
    def _(ci):
        pltpu.sync_copy(ones, dacc.at[dstb.at[ci, 0]], add=True)

    @pl.when(wid < _DEG_EXTRA)
    def _():
        pltpu.sync_copy(
            dst_hbm.at[pl.ds(NCHUNK + NC * NS * _DEG_CHUNKS + wid, 1)],
            dstb.at[pl.ds(0, 1)])
        pltpu.sync_copy(ones, dacc.at[dstb.at[0, 0]], add=True)

    plsc.subcore_barrier()
    pltpu.sync_copy(
        dacc.at[pl.ds(sid * RPT, RPT)], out_hbm.at[cid, pl.ds(sid * RPT, RPT)]
    )


_AGG_CHUNKS = NCHUNK // NS
_AGG_BATCH = 26
_AGG_NB = _AGG_CHUNKS // _AGG_BATCH
_AGG_EXTRA = NCHUNK - _AGG_CHUNKS * NS


@functools.partial(
    pl.kernel,
    mesh=_MESH,
    out_type=jax.ShapeDtypeStruct((NC, NPAD, DH), jnp.float32),
    scratch_types=[
        pltpu.VMEM((_AGG_BATCH * CHUNK,), jnp.int32),
        pltpu.VMEM((_AGG_BATCH, 1, CHUNK), jnp.int32),
        pltpu.VMEM((CHUNK, DH), jnp.float32),
        pltpu.VMEM((CHUNK, DH), jnp.float32),
        pltpu.VMEM_SHARED((NPAD, DH), jnp.float32),
        pltpu.SemaphoreType.DMA,
        pltpu.SemaphoreType.DMA,
    ],
)
def _agg_kernel(hp_hbm, src_hbm, dst_hbm, out_hbm,
                srcb, dstb, rows0, rows1, acc, sem0, sem1):
    cid = lax.axis_index("c")
    sid = lax.axis_index("s")

    _zero_fill(rows0, 64)

    @pl.loop(0, RPT // 64)
    def _(k):
        pltpu.async_copy(rows0.at[pl.ds(0, 64)],
                         acc.at[pl.ds(sid * RPT + k * 64, 64)], sem0)

    @pl.loop(0, RPT // 64)
    def _(k):
        pltpu.make_async_copy(rows0.at[pl.ds(0, 64)],
                              acc.at[pl.ds(sid * RPT, 64)], sem0).wait()

    plsc.subcore_barrier()

    off = cid * NPAD

    @pl.loop(0, _AGG_NB)
    def _(b):
        e0 = (sid * _AGG_CHUNKS + b * _AGG_BATCH) * CHUNK
        pltpu.sync_copy(src_hbm.at[pl.ds(e0, _AGG_BATCH * CHUNK)], srcb)
        pltpu.sync_copy(
            dst_hbm.at[pl.ds(NCHUNK + sid * _AGG_CHUNKS + b * _AGG_BATCH,
                             _AGG_BATCH)],
            dstb)

        @pl.loop(0, _AGG_BATCH * CHUNK // 16)
        def _(j):
            srcb[pl.ds(j * 16, 16)] = srcb[pl.ds(j * 16, 16)] + off

        pltpu.async_copy(hp_hbm.at[srcb.at[pl.ds(0, CHUNK)]], rows0, sem0)

        @pl.loop(0, _AGG_BATCH, step=2)
        def _(ci):
            pltpu.async_copy(
                hp_hbm.at[srcb.at[pl.ds((ci + 1) * CHUNK, CHUNK)]],
                rows1, sem1)
            pltpu.make_async_copy(
                hp_hbm.at[srcb.at[pl.ds(0, CHUNK)]], rows0, sem0).wait()
            pltpu.sync_copy(rows0, acc.at[dstb.at[ci, 0]], add=True)

            @pl.when(ci + 2 < _AGG_BATCH)
            def _():
                pltpu.async_copy(
                    hp_hbm.at[srcb.at[pl.ds((ci + 2) * CHUNK, CHUNK)]],
                    rows0, sem0)

            pltpu.make_async_copy(
                hp_hbm.at[srcb.at[pl.ds(0, CHUNK)]], rows1, sem1).wait()
            pltpu.sync_copy(rows1, acc.at[dstb.at[ci + 1, 0]], add=True)

    @pl.when(sid < _AGG_EXTRA)
    def _():
        ce = NS * _AGG_CHUNKS + sid
        pltpu.sync_copy(src_hbm.at[pl.ds(ce * CHUNK, CHUNK)],
                        srcb.at[pl.ds(0, CHUNK)])
        pltpu.sync_copy(dst_hbm.at[pl.ds(NCHUNK + ce, 1)],
                        dstb.at[pl.ds(0, 1)])

        @pl.loop(0, CHUNK // 16)
        def _(j):
            srcb[pl.ds(j * 16, 16)] = srcb[pl.ds(j * 16, 16)] + off

        pltpu.async_copy(hp_hbm.at[srcb.at[pl.ds(0, CHUNK)]],
                         rows0, sem0).wait()
        pltpu.sync_copy(rows0, acc.at[dstb.at[0, 0]], add=True)

    plsc.subcore_barrier()
    pltpu.sync_copy(
        acc.at[pl.ds(sid * RPT, RPT)], out_hbm.at[cid, pl.ds(sid * RPT, RPT)]
    )


def _mm_body(x_ref, w_ref, o_ref):
    o_ref[0] = jnp.dot(x_ref[...].astype(jnp.bfloat16),
                       w_ref[...].astype(jnp.bfloat16),
                       preferred_element_type=jnp.float32)


def _matmul_stacked(x, w):
    return pl.pallas_call(
        _mm_body,
        grid=(GRID_R, NC),
        in_specs=[
            pl.BlockSpec((ROWB, D), lambda i, j: (i, 0)),
            pl.BlockSpec((D, DH), lambda i, j: (0, j)),
        ],
        out_specs=pl.BlockSpec((1, ROWB, DH), lambda i, j: (j, i, 0)),
        out_shape=jax.ShapeDtypeStruct((NC, NPAD, DH), jnp.float32),
    )(x, w)


def _dinv_of(p0_ref, p1_ref):
    deg = p0_ref[0, :, :1] + p1_ref[0, :, :1] + 1.0
    return lax.rsqrt(deg)


def _scale_body(h_ref, p0_ref, p1_ref, o_ref):
    o_ref[0] = h_ref[0] * _dinv_of(p0_ref, p1_ref)


def _scale_stacked(h, degp):
    return pl.pallas_call(
        _scale_body,
        grid=(GRID_R, NC),
        in_specs=[
            pl.BlockSpec((1, ROWB, DH), lambda i, j: (j, i, 0)),
            pl.BlockSpec((1, ROWB, 16), lambda i, j: (0, i, 0)),
            pl.BlockSpec((1, ROWB, 16), lambda i, j: (1, i, 0)),
        ],
        out_specs=pl.BlockSpec((1, ROWB, DH), lambda i, j: (j, i, 0)),
        out_shape=jax.ShapeDtypeStruct((NC, NPAD, DH), jnp.float32),
    )(h, degp, degp)


def _layer2_body(a0_ref, a1_ref, h0_ref, h1_ref, p0_ref, p1_ref,
                 b_ref, w_ref, o_ref):
    dinv = _dinv_of(p0_ref, p1_ref)
    a0 = a0_ref[0].astype(jnp.float32) + h0_ref[0].astype(jnp.float32)
    a1 = a1_ref[0].astype(jnp.float32) + h1_ref[0].astype(jnp.float32)
    z0 = jnp.maximum(dinv * a0 + b_ref[:, :DH], 0.0)
    z1 = jnp.maximum(dinv * a1 + b_ref[:, DH:], 0.0)
    z = jnp.concatenate([z0, z1], axis=1).astype(jnp.bfloat16)
    o_ref[0] = dinv * jnp.dot(z, w_ref[...].astype(jnp.bfloat16),
                              preferred_element_type=jnp.float32)


def _layer2(acc1, hp1, degp, b1, w2):
    half = pl.BlockSpec((1, ROWB, DH), lambda i, j: (0, i, 0))
    half1 = pl.BlockSpec((1, ROWB, DH), lambda i, j: (1, i, 0))
    return pl.pallas_call(
        _layer2_body,
        grid=(GRID_R, NC),
        in_specs=[
            half, half1, half, half1,
            pl.BlockSpec((1, ROWB, 16), lambda i, j: (0, i, 0)),
            pl.BlockSpec((1, ROWB, 16), lambda i, j: (1, i, 0)),
            pl.BlockSpec((1, D), lambda i, j: (0, 0)),
            pl.BlockSpec((D, DH), lambda i, j: (0, j)),
        ],
        out_specs=pl.BlockSpec((1, ROWB, DH), lambda i, j: (j, i, 0)),
        out_shape=jax.ShapeDtypeStruct((NC, NPAD, DH), jnp.float32),
    )(acc1, acc1, hp1, hp1, degp, degp, b1, w2)


def _final_body(a_ref, h_ref, p0_ref, p1_ref, b_ref, o_ref):
    dinv = _dinv_of(p0_ref, p1_ref)
    s = a_ref[0].astype(jnp.float32) + h_ref[0].astype(jnp.float32)
    o_ref[...] = dinv * s + b_ref[0]


def _final(acc2, hp2, degp, b2):
    return pl.pallas_call(
        _final_body,
        grid=(GRID_R, NC),
        in_specs=[
            pl.BlockSpec((1, ROWB, DH), lambda i, j: (j, i, 0)),
            pl.BlockSpec((1, ROWB, DH), lambda i, j: (j, i, 0)),
            pl.BlockSpec((1, ROWB, 16), lambda i, j: (0, i, 0)),
            pl.BlockSpec((1, ROWB, 16), lambda i, j: (1, i, 0)),
            pl.BlockSpec((1, 1, DH), lambda i, j: (j, 0, 0)),
        ],
        out_specs=pl.BlockSpec((ROWB, DH), lambda i, j: (i, j)),
        out_shape=jax.ShapeDtypeStruct((N, D), jnp.float32),
    )(acc2, hp2, degp, degp, b2)


def kernel(x, edge_index, W1, b1, W2, b2):
    ei = edge_index.astype(jnp.int32)
    src1d = ei.reshape(2 * E)
    dst3d = ei.reshape(2 * NCHUNK, 1, CHUNK)
    b1r = b1.reshape(1, D)
    b2r = b2.reshape(NC, 1, DH)

    h1 = _matmul_stacked(x, W1)
    degp = _deg_kernel(dst3d)
    hp1 = _scale_stacked(h1, degp)
    acc1 = _agg_kernel(hp1.reshape(NC * NPAD, DH), src1d, dst3d)
    hp2 = _layer2(acc1, hp1, degp, b1r, W2)
    acc2 = _agg_kernel(hp2.reshape(NC * NPAD, DH), src1d, dst3d)
    return _final(acc2, hp2, degp, b2r)


# speedup vs baseline: 19.5338x; 1.0479x over previous
import functools

import jax
import jax.numpy as jnp
from jax import lax
from jax.experimental import pallas as pl
from jax.experimental.pallas import tpu as pltpu
from jax.experimental.pallas import tpu_sc as plsc

N = 10000
D = 256
DH = 128
E = 160000
NC = 2
NS = 16
CHUNK = 128
NCHUNK = E // CHUNK
NPAD = 10240
RPT = NPAD // NS
ROWB = 2000
GRID_R = N // ROWB

_MESH = plsc.VectorSubcoreMesh(core_axis_name="c", subcore_axis_name="s")


def _zero_fill(buf, rows):
    width = buf.shape[1]
    if buf.dtype == jnp.bfloat16:
        z = jnp.zeros((2, 16), jnp.bfloat16)

        @pl.loop(0, rows // 2)
        def _(r):
            r2 = pl.multiple_of(r * 2, 2)

            @pl.loop(0, width // 16)
            def _(c):
                buf[pl.ds(r2, 2), pl.ds(c * 16, 16)] = z
    else:
        z = jnp.zeros((16,), buf.dtype)

        @pl.loop(0, rows)
        def _(r):
            @pl.loop(0, width // 16)
            def _(c):
                buf[r, pl.ds(c * 16, 16)] = z


_DEG_CHUNKS = NCHUNK // (NC * NS)
_DEG_EXTRA = NCHUNK - _DEG_CHUNKS * NC * NS


@functools.partial(
    pl.kernel,
    mesh=_MESH,
    out_type=jax.ShapeDtypeStruct((NC, NPAD, 16), jnp.float32),
    scratch_types=[
        pltpu.VMEM((_DEG_CHUNKS, 1, CHUNK), jnp.int32),
        pltpu.VMEM((CHUNK, 16), jnp.float32),
        pltpu.VMEM((64, 16), jnp.float32),
        pltpu.VMEM_SHARED((NPAD, 16), jnp.float32),
        pltpu.SemaphoreType.DMA,
    ],
)
def _deg_kernel(dst_hbm, out_hbm, dstb, ones, zbuf, dacc, zsem):
    cid = lax.axis_index("c")
    sid = lax.axis_index("s")

    _zero_fill(zbuf, 64)
    o = jnp.ones((16,), jnp.float32)

    @pl.loop(0, CHUNK)
    def _(r):
        ones[r, pl.ds(0, 16)] = o

    @pl.loop(0, RPT // 64)
    def _(k):
        pltpu.async_copy(zbuf, dacc.at[pl.ds(sid * RPT + k * 64, 64)], zsem)

    @pl.loop(0, RPT // 64)
    def _(k):
        pltpu.make_async_copy(zbuf, dacc.at[pl.ds(sid * RPT, 64)],
                              zsem).wait()

    wid = cid * NS + sid
    row0 = NCHUNK + wid * _DEG_CHUNKS
    pltpu.sync_copy(dst_hbm.at[pl.ds(row0, _DEG_CHUNKS)], dstb)

    plsc.subcore_barrier()

    @pl.loop(0, _DEG_CHUNKS)
    def _(ci):
        pltpu.sync_copy(ones, dacc.at[dstb.at[ci, 0]], add=True)

    @pl.when(wid < _DEG_EXTRA)
    def _():
        pltpu.sync_copy(
            dst_hbm.at[pl.ds(NCHUNK + NC * NS * _DEG_CHUNKS + wid, 1)],
            dstb.at[pl.ds(0, 1)])
        pltpu.sync_copy(ones, dacc.at[dstb.at[0, 0]], add=True)

    plsc.subcore_barrier()
    pltpu.sync_copy(
        dacc.at[pl.ds(sid * RPT, RPT)], out_hbm.at[cid, pl.ds(sid * RPT, RPT)]
    )


_AGG_CHUNKS = NCHUNK // NS
_AGG_BATCH = 26
_AGG_NB = _AGG_CHUNKS // _AGG_BATCH
_AGG_EXTRA = NCHUNK - _AGG_CHUNKS * NS


@functools.partial(
    pl.kernel,
    mesh=_MESH,
    out_type=jax.ShapeDtypeStruct((NC, NPAD, DH), jnp.float32),
    scratch_types=[
        pltpu.VMEM((_AGG_BATCH * CHUNK,), jnp.int32),
        pltpu.VMEM((_AGG_BATCH, 1, CHUNK), jnp.int32),
        pltpu.VMEM((CHUNK, DH), jnp.float32),
        pltpu.VMEM((CHUNK, DH), jnp.float32),
        pltpu.VMEM_SHARED((NPAD, DH), jnp.float32),
        pltpu.SemaphoreType.DMA,
        pltpu.SemaphoreType.DMA,
    ],
)
def _agg_kernel(hp_hbm, src_hbm, dst_hbm, out_hbm,
                srcb, dstb, rows0, rows1, acc, sem0, sem1):
    cid = lax.axis_index("c")
    sid = lax.axis_index("s")

    _zero_fill(rows0, 64)

    @pl.loop(0, RPT // 64)
    def _(k):
        pltpu.async_copy(rows0.at[pl.ds(0, 64)],
                         acc.at[pl.ds(sid * RPT + k * 64, 64)], sem0)

    @pl.loop(0, RPT // 64)
    def _(k):
        pltpu.make_async_copy(rows0.at[pl.ds(0, 64)],
                              acc.at[pl.ds(sid * RPT, 64)], sem0).wait()

    plsc.subcore_barrier()

    off = cid * NPAD

    @pl.loop(0, _AGG_NB)
    def _(b):
        e0 = (sid * _AGG_CHUNKS + b * _AGG_BATCH) * CHUNK
        pltpu.sync_copy(src_hbm.at[pl.ds(e0, _AGG_BATCH * CHUNK)], srcb)
        pltpu.sync_copy(
            dst_hbm.at[pl.ds(NCHUNK + sid * _AGG_CHUNKS + b * _AGG_BATCH,
                             _AGG_BATCH)],
            dstb)

        @pl.loop(0, _AGG_BATCH * CHUNK // 16)
        def _(j):
            srcb[pl.ds(j * 16, 16)] = srcb[pl.ds(j * 16, 16)] + off

        pltpu.async_copy(hp_hbm.at[srcb.at[pl.ds(0, CHUNK)]], rows0, sem0)

        @pl.loop(0, _AGG_BATCH, step=2)
        def _(ci):
            pltpu.async_copy(
                hp_hbm.at[srcb.at[pl.ds((ci + 1) * CHUNK, CHUNK)]],
                rows1, sem1)
            pltpu.make_async_copy(
                hp_hbm.at[srcb.at[pl.ds(0, CHUNK)]], rows0, sem0).wait()
            pltpu.sync_copy(rows0, acc.at[dstb.at[ci, 0]], add=True)

            @pl.when(ci + 2 < _AGG_BATCH)
            def _():
                pltpu.async_copy(
                    hp_hbm.at[srcb.at[pl.ds((ci + 2) * CHUNK, CHUNK)]],
                    rows0, sem0)

            pltpu.make_async_copy(
                hp_hbm.at[srcb.at[pl.ds(0, CHUNK)]], rows1, sem1).wait()
            pltpu.sync_copy(rows1, acc.at[dstb.at[ci + 1, 0]], add=True)

    @pl.when(sid < _AGG_EXTRA)
    def _():
        ce = NS * _AGG_CHUNKS + sid
        pltpu.sync_copy(src_hbm.at[pl.ds(ce * CHUNK, CHUNK)],
                        srcb.at[pl.ds(0, CHUNK)])
        pltpu.sync_copy(dst_hbm.at[pl.ds(NCHUNK + ce, 1)],
                        dstb.at[pl.ds(0, 1)])

        @pl.loop(0, CHUNK // 16)
        def _(j):
            srcb[pl.ds(j * 16, 16)] = srcb[pl.ds(j * 16, 16)] + off

        pltpu.async_copy(hp_hbm.at[srcb.at[pl.ds(0, CHUNK)]],
                         rows0, sem0).wait()
        pltpu.sync_copy(rows0, acc.at[dstb.at[0, 0]], add=True)

    plsc.subcore_barrier()
    pltpu.sync_copy(
        acc.at[pl.ds(sid * RPT, RPT)], out_hbm.at[cid, pl.ds(sid * RPT, RPT)]
    )


_STACK_SPEC = pl.BlockSpec((NC, ROWB, DH), lambda i: (0, i, 0))
_DEG0_SPEC = pl.BlockSpec((1, ROWB, 16), lambda i: (0, i, 0))
_DEG1_SPEC = pl.BlockSpec((1, ROWB, 16), lambda i: (1, i, 0))
_STACK_TY = jax.ShapeDtypeStruct((NC, NPAD, DH), jnp.float32)


def _mm_body(x_ref, w_ref, o_ref):
    h = jnp.dot(x_ref[...].astype(jnp.bfloat16),
                w_ref[...].astype(jnp.bfloat16),
                preferred_element_type=jnp.float32)
    o_ref[0] = h[:, :DH]
    o_ref[1] = h[:, DH:]


def _matmul_stacked(x, w):
    return pl.pallas_call(
        _mm_body,
        grid=(GRID_R,),
        in_specs=[
            pl.BlockSpec((ROWB, D), lambda i: (i, 0)),
            pl.BlockSpec((D, D), lambda i: (0, 0)),
        ],
        out_specs=_STACK_SPEC,
        out_shape=_STACK_TY,
    )(x, w)


def _dinv_of(p0_ref, p1_ref):
    deg = p0_ref[0, :, :1] + p1_ref[0, :, :1] + 1.0
    return lax.rsqrt(deg)


def _scale_body(h_ref, p0_ref, p1_ref, o_ref):
    dinv = _dinv_of(p0_ref, p1_ref)
    o_ref[0] = h_ref[0] * dinv
    o_ref[1] = h_ref[1] * dinv


def _scale_stacked(h, degp):
    return pl.pallas_call(
        _scale_body,
        grid=(GRID_R,),
        in_specs=[_STACK_SPEC, _DEG0_SPEC, _DEG1_SPEC],
        out_specs=_STACK_SPEC,
        out_shape=_STACK_TY,
    )(h, degp, degp)


def _layer2_body(a_ref, h_ref, p0_ref, p1_ref, b_ref, w_ref, o_ref):
    dinv = _dinv_of(p0_ref, p1_ref)
    z0 = jnp.maximum(dinv * (a_ref[0] + h_ref[0]) + b_ref[:, :DH], 0.0)
    z1 = jnp.maximum(dinv * (a_ref[1] + h_ref[1]) + b_ref[:, DH:], 0.0)
    z = jnp.concatenate([z0, z1], axis=1).astype(jnp.bfloat16)
    r = jnp.dot(z, w_ref[...].astype(jnp.bfloat16),
                preferred_element_type=jnp.float32)
    o_ref[0] = dinv * r[:, :DH]
    o_ref[1] = dinv * r[:, DH:]


def _layer2(acc1, hp1, degp, b1, w2):
    return pl.pallas_call(
        _layer2_body,
        grid=(GRID_R,),
        in_specs=[
            _STACK_SPEC, _STACK_SPEC, _DEG0_SPEC, _DEG1_SPEC,
            pl.BlockSpec((1, D), lambda i: (0, 0)),
            pl.BlockSpec((D, D), lambda i: (0, 0)),
        ],
        out_specs=_STACK_SPEC,
        out_shape=_STACK_TY,
    )(acc1, hp1, degp, degp, b1, w2)


def _final_body(a_ref, h_ref, p0_ref, p1_ref, b_ref, o_ref):
    dinv = _dinv_of(p0_ref, p1_ref)
    o0 = dinv * (a_ref[0] + h_ref[0]) + b_ref[:, :DH]
    o1 = dinv * (a_ref[1] + h_ref[1]) + b_ref[:, DH:]
    o_ref[...] = jnp.concatenate([o0, o1], axis=1)


def _final(acc2, hp2, degp, b2):
    return pl.pallas_call(
        _final_body,
        grid=(GRID_R,),
        in_specs=[
            _STACK_SPEC, _STACK_SPEC, _DEG0_SPEC, _DEG1_SPEC,
            pl.BlockSpec((1, D), lambda i: (0, 0)),
        ],
        out_specs=pl.BlockSpec((ROWB, D), lambda i: (i, 0)),
        out_shape=jax.ShapeDtypeStruct((N, D), jnp.float32),
    )(acc2, hp2, degp, degp, b2)


def kernel(x, edge_index, W1, b1, W2, b2):
    ei = edge_index.astype(jnp.int32)
    src1d = ei.reshape(2 * E)
    dst3d = ei.reshape(2 * NCHUNK, 1, CHUNK)
    b1r = b1.reshape(1, D)
    b2r = b2.reshape(1, D)

    h1 = _matmul_stacked(x, W1)
    degp = _deg_kernel(dst3d)
    hp1 = _scale_stacked(h1, degp)
    acc1 = _agg_kernel(hp1.reshape(NC * NPAD, DH), src1d, dst3d)
    hp2 = _layer2(acc1, hp1, degp, b1r, W2)
    acc2 = _agg_kernel(hp2.reshape(NC * NPAD, DH), src1d, dst3d)
    return _final(acc2, hp2, degp, b2r)


# speedup vs baseline: 19.8969x; 1.0186x over previous
import functools

import jax
import jax.numpy as jnp
from jax import lax
from jax.experimental import pallas as pl
from jax.experimental.pallas import tpu as pltpu
from jax.experimental.pallas import tpu_sc as plsc

N = 10000
D = 256
DH = 128
E = 160000
NC = 2
NS = 16
CHUNK = 128
NCHUNK = E // CHUNK
NPAD = 10240
RPT = NPAD // NS
ROWB = 2000
GRID_R = N // ROWB

_MESH = plsc.VectorSubcoreMesh(core_axis_name="c", subcore_axis_name="s")


def _zero_fill(buf, rows):
    width = buf.shape[1]
    if buf.dtype == jnp.bfloat16:
        z = jnp.zeros((2, 16), jnp.bfloat16)

        @pl.loop(0, rows // 2)
        def _(r):
            r2 = pl.multiple_of(r * 2, 2)

            @pl.loop(0, width // 16)
            def _(c):
                buf[pl.ds(r2, 2), pl.ds(c * 16, 16)] = z
    else:
        z = jnp.zeros((16,), buf.dtype)

        @pl.loop(0, rows)
        def _(r):
            @pl.loop(0, width // 16)
            def _(c):
                buf[r, pl.ds(c * 16, 16)] = z


_DEG_CHUNKS = NCHUNK // (NC * NS)
_DEG_EXTRA = NCHUNK - _DEG_CHUNKS * NC * NS


@functools.partial(
    pl.kernel,
    mesh=_MESH,
    out_type=jax.ShapeDtypeStruct((NC, NPAD, 16), jnp.float32),
    scratch_types=[
        pltpu.VMEM((_DEG_CHUNKS, 1, CHUNK), jnp.int32),
        pltpu.VMEM((CHUNK, 16), jnp.float32),
        pltpu.VMEM((64, 16), jnp.float32),
        pltpu.VMEM_SHARED((NPAD, 16), jnp.float32),
        pltpu.SemaphoreType.DMA,
    ],
)
def _deg_kernel(dst_hbm, out_hbm, dstb, ones, zbuf, dacc, zsem):
    cid = lax.axis_index("c")
    sid = lax.axis_index("s")

    _zero_fill(zbuf, 64)
    o = jnp.ones((16,), jnp.float32)

    @pl.loop(0, CHUNK)
    def _(r):
        ones[r, pl.ds(0, 16)] = o

    @pl.loop(0, RPT // 64)
    def _(k):
        pltpu.async_copy(zbuf, dacc.at[pl.ds(sid * RPT + k * 64, 64)], zsem)

    @pl.loop(0, RPT // 64)
    def _(k):
        pltpu.make_async_copy(zbuf, dacc.at[pl.ds(sid * RPT, 64)],
                              zsem).wait()

    wid = cid * NS + sid
    row0 = NCHUNK + wid * _DEG_CHUNKS
    pltpu.sync_copy(dst_hbm.at[pl.ds(row0, _DEG_CHUNKS)], dstb)

    plsc.subcore_barrier()

    @pl.loop(0, _DEG_CHUNKS)
    def _(ci):
        pltpu.sync_copy(ones, dacc.at[dstb.at[ci, 0]], add=True)

    @pl.when(wid < _DEG_EXTRA)
    def _():
        pltpu.sync_copy(
            dst_hbm.at[pl.ds(NCHUNK + NC * NS * _DEG_CHUNKS + wid, 1)],
            dstb.at[pl.ds(0, 1)])
        pltpu.sync_copy(ones, dacc.at[dstb.at[0, 0]], add=True)

    plsc.subcore_barrier()
    pltpu.sync_copy(
        dacc.at[pl.ds(sid * RPT, RPT)], out_hbm.at[cid, pl.ds(sid * RPT, RPT)]
    )


_AGG_CHUNKS = NCHUNK // NS
_AGG_BATCH = 26
_AGG_NB = _AGG_CHUNKS // _AGG_BATCH
_AGG_EXTRA = NCHUNK - _AGG_CHUNKS * NS


@functools.partial(
    pl.kernel,
    mesh=_MESH,
    out_type=jax.ShapeDtypeStruct((NC, NPAD, DH), jnp.float32),
    scratch_types=[
        pltpu.VMEM((_AGG_BATCH * CHUNK,), jnp.int32),
        pltpu.VMEM((_AGG_BATCH, 1, CHUNK), jnp.int32),
        pltpu.VMEM((CHUNK, DH), jnp.float32),
        pltpu.VMEM((CHUNK, DH), jnp.float32),
        pltpu.VMEM_SHARED((NPAD, DH), jnp.float32),
        pltpu.SemaphoreType.DMA,
        pltpu.SemaphoreType.DMA,
    ],
)
def _agg_kernel(hp_hbm, src_hbm, dst_hbm, out_hbm,
                srcb, dstb, rows0, rows1, acc, sem0, sem1):
    cid = lax.axis_index("c")
    sid = lax.axis_index("s")

    _zero_fill(rows0, 64)

    @pl.loop(0, RPT // 64)
    def _(k):
        pltpu.async_copy(rows0.at[pl.ds(0, 64)],
                         acc.at[pl.ds(sid * RPT + k * 64, 64)], sem0)

    off = cid * NPAD

    def _load_batch(b):
        e0 = (sid * _AGG_CHUNKS + b * _AGG_BATCH) * CHUNK
        pltpu.sync_copy(src_hbm.at[pl.ds(e0, _AGG_BATCH * CHUNK)], srcb)
        pltpu.sync_copy(
            dst_hbm.at[pl.ds(NCHUNK + sid * _AGG_CHUNKS + b * _AGG_BATCH,
                             _AGG_BATCH)],
            dstb)

        @pl.loop(0, _AGG_BATCH * CHUNK // 16)
        def _(j):
            srcb[pl.ds(j * 16, 16)] = srcb[pl.ds(j * 16, 16)] + off

    _load_batch(0)

    @pl.loop(0, RPT // 64)
    def _(k):
        pltpu.make_async_copy(rows0.at[pl.ds(0, 64)],
                              acc.at[pl.ds(sid * RPT, 64)], sem0).wait()

    plsc.subcore_barrier()

    @pl.loop(0, _AGG_NB)
    def _(b):
        @pl.when(b > 0)
        def _():
            _load_batch(b)

        pltpu.async_copy(hp_hbm.at[srcb.at[pl.ds(0, CHUNK)]], rows0, sem0)

        @pl.loop(0, _AGG_BATCH, step=2)
        def _(ci):
            pltpu.async_copy(
                hp_hbm.at[srcb.at[pl.ds((ci + 1) * CHUNK, CHUNK)]],
                rows1, sem1)
            pltpu.make_async_copy(
                hp_hbm.at[srcb.at[pl.ds(0, CHUNK)]], rows0, sem0).wait()
            pltpu.sync_copy(rows0, acc.at[dstb.at[ci, 0]], add=True)

            @pl.when(ci + 2 < _AGG_BATCH)
            def _():
                pltpu.async_copy(
                    hp_hbm.at[srcb.at[pl.ds((ci + 2) * CHUNK, CHUNK)]],
                    rows0, sem0)

            pltpu.make_async_copy(
                hp_hbm.at[srcb.at[pl.ds(0, CHUNK)]], rows1, sem1).wait()
            pltpu.sync_copy(rows1, acc.at[dstb.at[ci + 1, 0]], add=True)

    @pl.when(sid < _AGG_EXTRA)
    def _():
        ce = NS * _AGG_CHUNKS + sid
        pltpu.sync_copy(src_hbm.at[pl.ds(ce * CHUNK, CHUNK)],
                        srcb.at[pl.ds(0, CHUNK)])
        pltpu.sync_copy(dst_hbm.at[pl.ds(NCHUNK + ce, 1)],
                        dstb.at[pl.ds(0, 1)])

        @pl.loop(0, CHUNK // 16)
        def _(j):
            srcb[pl.ds(j * 16, 16)] = srcb[pl.ds(j * 16, 16)] + off

        pltpu.async_copy(hp_hbm.at[srcb.at[pl.ds(0, CHUNK)]],
                         rows0, sem0).wait()
        pltpu.sync_copy(rows0, acc.at[dstb.at[0, 0]], add=True)

    plsc.subcore_barrier()
    pltpu.sync_copy(
        acc.at[pl.ds(sid * RPT, RPT)], out_hbm.at[cid, pl.ds(sid * RPT, RPT)]
    )


_STACK_SPEC = pl.BlockSpec((NC, ROWB, DH), lambda i: (0, i, 0))
_DEG0_SPEC = pl.BlockSpec((1, ROWB, 16), lambda i: (0, i, 0))
_DEG1_SPEC = pl.BlockSpec((1, ROWB, 16), lambda i: (1, i, 0))
_STACK_TY = jax.ShapeDtypeStruct((NC, NPAD, DH), jnp.float32)


def _mm_body(x_ref, w_ref, o_ref):
    h = jnp.dot(x_ref[...].astype(jnp.bfloat16),
                w_ref[...].astype(jnp.bfloat16),
                preferred_element_type=jnp.float32)
    o_ref[0] = h[:, :DH]
    o_ref[1] = h[:, DH:]


def _matmul_stacked(x, w):
    return pl.pallas_call(
        _mm_body,
        grid=(GRID_R,),
        in_specs=[
            pl.BlockSpec((ROWB, D), lambda i: (i, 0)),
            pl.BlockSpec((D, D), lambda i: (0, 0)),
        ],
        out_specs=_STACK_SPEC,
        out_shape=_STACK_TY,
    )(x, w)


def _dinv_of(p0_ref, p1_ref):
    deg = p0_ref[0, :, :1] + p1_ref[0, :, :1] + 1.0
    return lax.rsqrt(deg)


def _scale_body(h_ref, p0_ref, p1_ref, o_ref):
    dinv = _dinv_of(p0_ref, p1_ref)
    o_ref[0] = h_ref[0] * dinv
    o_ref[1] = h_ref[1] * dinv


def _scale_stacked(h, degp):
    return pl.pallas_call(
        _scale_body,
        grid=(GRID_R,),
        in_specs=[_STACK_SPEC, _DEG0_SPEC, _DEG1_SPEC],
        out_specs=_STACK_SPEC,
        out_shape=_STACK_TY,
    )(h, degp, degp)


def _layer2_body(a_ref, h_ref, p0_ref, p1_ref, b_ref, w_ref, o_ref):
    dinv = _dinv_of(p0_ref, p1_ref)
    z0 = jnp.maximum(dinv * (a_ref[0] + h_ref[0]) + b_ref[:, :DH], 0.0)
    z1 = jnp.maximum(dinv * (a_ref[1] + h_ref[1]) + b_ref[:, DH:], 0.0)
    z = jnp.concatenate([z0, z1], axis=1).astype(jnp.bfloat16)
    r = jnp.dot(z, w_ref[...].astype(jnp.bfloat16),
                preferred_element_type=jnp.float32)
    o_ref[0] = dinv * r[:, :DH]
    o_ref[1] = dinv * r[:, DH:]


def _layer2(acc1, hp1, degp, b1, w2):
    return pl.pallas_call(
        _layer2_body,
        grid=(GRID_R,),
        in_specs=[
            _STACK_SPEC, _STACK_SPEC, _DEG0_SPEC, _DEG1_SPEC,
            pl.BlockSpec((1, D), lambda i: (0, 0)),
            pl.BlockSpec((D, D), lambda i: (0, 0)),
        ],
        out_specs=_STACK_SPEC,
        out_shape=_STACK_TY,
    )(acc1, hp1, degp, degp, b1, w2)


def _final_body(a_ref, h_ref, p0_ref, p1_ref, b_ref, o_ref):
    dinv = _dinv_of(p0_ref, p1_ref)
    o0 = dinv * (a_ref[0] + h_ref[0]) + b_ref[:, :DH]
    o1 = dinv * (a_ref[1] + h_ref[1]) + b_ref[:, DH:]
    o_ref[...] = jnp.concatenate([o0, o1], axis=1)


def _final(acc2, hp2, degp, b2):
    return pl.pallas_call(
        _final_body,
        grid=(GRID_R,),
        in_specs=[
            _STACK_SPEC, _STACK_SPEC, _DEG0_SPEC, _DEG1_SPEC,
            pl.BlockSpec((1, D), lambda i: (0, 0)),
        ],
        out_specs=pl.BlockSpec((ROWB, D), lambda i: (i, 0)),
        out_shape=jax.ShapeDtypeStruct((N, D), jnp.float32),
    )(acc2, hp2, degp, degp, b2)


def kernel(x, edge_index, W1, b1, W2, b2):
    ei = edge_index.astype(jnp.int32)
    src1d = ei.reshape(2 * E)
    dst3d = ei.reshape(2 * NCHUNK, 1, CHUNK)
    b1r = b1.reshape(1, D)
    b2r = b2.reshape(1, D)

    h1 = _matmul_stacked(x, W1)
    degp = _deg_kernel(dst3d)
    hp1 = _scale_stacked(h1, degp)
    acc1 = _agg_kernel(hp1.reshape(NC * NPAD, DH), src1d, dst3d)
    hp2 = _layer2(acc1, hp1, degp, b1r, W2)
    acc2 = _agg_kernel(hp2.reshape(NC * NPAD, DH), src1d, dst3d)
    return _final(acc2, hp2, degp, b2r)
